# baseline (speedup 1.0000x reference)
import functools

import jax
import jax.numpy as jnp
from jax import lax
from jax.experimental import pallas as pl
from jax.experimental.pallas import tpu as pltpu
from jax.experimental.pallas import tpu_sc as plsc

B = 4096
V = 100000
D = 64
L = 50
R = 5

NEG = -1e30

NC = 2
NS = 16
NW = NC * NS
PB = B // NW

VB = 1000
MS = 16
CH = 4


def _prep_body(feat_ref, mask_ref, r_ref, wgv_ref, bgv_ref, rate_ref,
               fp_ref, rpp_ref, msum_ref, rext_ref):
    f = feat_ref[...]
    wt = wgv_ref[:D, :]
    wb = wgv_ref[D:, :]
    fp_ref[...] = jnp.dot(f, wt, preferred_element_type=jnp.float32)
    m = mask_ref[...]
    rpp_ref[...] = jnp.where(m > 0, r_ref[...], R)
    s = jnp.sum(m.astype(jnp.float32), axis=1, keepdims=True)
    msum_ref[...] = jnp.broadcast_to(s, (VB, MS))
    rex = jnp.dot(rate_ref[...], wb, preferred_element_type=jnp.float32) + bgv_ref[...]
    row = lax.broadcasted_iota(jnp.int32, (8, D), 0)
    rext_ref[...] = jnp.where(row < R, rex, NEG)


def _combine_body(self_ref, acc_ref, msum_ref, w1_ref, b1_ref, out_ref):
    denom = jnp.maximum(msum_ref[:, :1], 1.0)
    neigh = acc_ref[...] / denom
    x = (jnp.dot(self_ref[...], w1_ref[:D, :], preferred_element_type=jnp.float32)
         + jnp.dot(neigh, w1_ref[D:, :], preferred_element_type=jnp.float32)
         + b1_ref[...])
    out_ref[...] = jnp.maximum(x, 0.0)


def _sc_body(nodes_hbm, uv_hbm, rpp_hbm, fp_hbm, rext_hbm, msum_hbm, feat_hbm,
             acc_out, self_out, msum_out,
             nodes_v, uv_rows, rp_rows, msum_rows, self_rows, neigh_buf,
             e_buf, rr_buf, sem_e, sem_r, sem_a, sem_b):
    wid = lax.axis_index("s") * NC + lax.axis_index("c")
    base = wid * PB
    pltpu.sync_copy(nodes_hbm.at[pl.ds(base, PB)], nodes_v)
    cp_uv = pltpu.async_copy(uv_hbm.at[nodes_v], uv_rows, sem_e)
    cp_rp = pltpu.async_copy(rpp_hbm.at[nodes_v], rp_rows, sem_r)
    cp_ms = pltpu.async_copy(msum_hbm.at[nodes_v], msum_rows, sem_a)
    cp_sf = pltpu.async_copy(feat_hbm.at[nodes_v], self_rows, sem_b)
    cp_uv.wait()
    cp_rp.wait()

    def fire(g, buf):
        for j in range(CH):
            i = g * CH + j
            pltpu.async_copy(fp_hbm.at[uv_rows.at[i]], e_buf.at[buf, j], sem_e)
            pltpu.async_copy(fp_hbm.at[uv_rows.at[i]], rr_buf.at[buf, j], sem_r)

    def drain(g, buf):
        for j in range(CH):
            i = g * CH + j
            pltpu.make_async_copy(fp_hbm.at[uv_rows.at[i]], e_buf.at[buf, j], sem_e).wait()
            pltpu.make_async_copy(fp_hbm.at[uv_rows.at[i]], rr_buf.at[buf, j], sem_r).wait()

    def compute(g, buf):
        for j in range(CH):
            def lbody(l, accs):
                out = []
                for c in range(4):
                    e = e_buf[buf, j, l, pl.ds(c * 16, 16)]
                    rr = rr_buf[buf, j, l, pl.ds(c * 16, 16)]
                    out.append(accs[c] + jnp.maximum(e + rr, 0.0))
                return tuple(out)
            z = jnp.zeros((16,), jnp.float32)
            accs = lax.fori_loop(0, L, lbody, (z, z, z, z))
            for c in range(4):
                neigh_buf[g * CH + j, pl.ds(c * 16, 16)] = accs[c]

    NG = PB // CH
    fire(0, 0)

    def step(g, _):
        buf = lax.rem(g, 2)
        drain(g, buf)

        @pl.when(g < NG - 1)
        def _():
            fire(g + 1, 1 - buf)

        compute(g, buf)
        return 0

    lax.fori_loop(0, NG, step, 0)

    pltpu.sync_copy(neigh_buf, acc_out.at[pl.ds(base, PB)])
    cp_sf.wait()
    pltpu.sync_copy(self_rows, self_out.at[pl.ds(base, PB)])
    cp_ms.wait()
    pltpu.sync_copy(msum_rows, msum_out.at[pl.ds(base, PB)])


def kernel(nodes, hist_uv, hist_uv_mask, hist_r, hist_r_mask,
           features, rating_emb, W_gv, b_gv, W1, b1):
    nodes = nodes.astype(jnp.int32)
    pad = ((0, 0), (0, D - L))
    uvp = jnp.pad(hist_uv.astype(jnp.int32), pad)
    hist_r = jnp.pad(hist_r.astype(jnp.int32), pad)
    mask = jnp.pad(hist_uv_mask.astype(jnp.int32), pad)
    rate_pad = jnp.zeros((8, D), jnp.float32).at[:R].set(rating_emb)
    bgv2 = b_gv.reshape(1, D)
    b12 = b1.reshape(1, D)

    grid1 = V // VB
    fp, rpp, msum, rext = pl.pallas_call(
        _prep_body,
        grid=(grid1,),
        in_specs=[
            pl.BlockSpec((VB, D), lambda i: (i, 0)),
            pl.BlockSpec((VB, D), lambda i: (i, 0)),
            pl.BlockSpec((VB, D), lambda i: (i, 0)),
            pl.BlockSpec((2 * D, D), lambda i: (0, 0)),
            pl.BlockSpec((1, D), lambda i: (0, 0)),
            pl.BlockSpec((8, D), lambda i: (0, 0)),
        ],
        out_specs=[
            pl.BlockSpec((VB, D), lambda i: (i, 0)),
            pl.BlockSpec((VB, D), lambda i: (i, 0)),
            pl.BlockSpec((VB, MS), lambda i: (i, 0)),
            pl.BlockSpec((8, D), lambda i: (0, 0)),
        ],
        out_shape=[
            jax.ShapeDtypeStruct((V, D), jnp.float32),
            jax.ShapeDtypeStruct((V, D), jnp.int32),
            jax.ShapeDtypeStruct((V, MS), jnp.float32),
            jax.ShapeDtypeStruct((8, D), jnp.float32),
        ],
    )(features, mask, hist_r, W_gv, bgv2, rate_pad)

    mesh = plsc.VectorSubcoreMesh(core_axis_name="c", subcore_axis_name="s")
    acc, selff, msumg = pl.kernel(
        _sc_body,
        out_type=[
            jax.ShapeDtypeStruct((B, D), jnp.float32),
            jax.ShapeDtypeStruct((B, D), jnp.float32),
            jax.ShapeDtypeStruct((B, MS), jnp.float32),
        ],
        mesh=mesh,
        compiler_params=pltpu.CompilerParams(use_tc_tiling_on_sc=False),
        scratch_types=[
            pltpu.VMEM((PB,), jnp.int32),
            pltpu.VMEM((PB, D), jnp.int32),
            pltpu.VMEM((PB, D), jnp.int32),
            pltpu.VMEM((PB, MS), jnp.float32),
            pltpu.VMEM((PB, D), jnp.float32),
            pltpu.VMEM((PB, D), jnp.float32),
            pltpu.VMEM((2, CH, D, D), jnp.float32),
            pltpu.VMEM((2, CH, D, D), jnp.float32),
            pltpu.SemaphoreType.DMA,
            pltpu.SemaphoreType.DMA,
            pltpu.SemaphoreType.DMA,
            pltpu.SemaphoreType.DMA,
        ],
    )(nodes, uvp, rpp, fp, rext, msum, features)

    BBLK = 512
    out = pl.pallas_call(
        _combine_body,
        grid=(B // BBLK,),
        in_specs=[
            pl.BlockSpec((BBLK, D), lambda i: (i, 0)),
            pl.BlockSpec((BBLK, D), lambda i: (i, 0)),
            pl.BlockSpec((BBLK, MS), lambda i: (i, 0)),
            pl.BlockSpec((2 * D, D), lambda i: (0, 0)),
            pl.BlockSpec((1, D), lambda i: (0, 0)),
        ],
        out_specs=pl.BlockSpec((BBLK, D), lambda i: (i, 0)),
        out_shape=jax.ShapeDtypeStruct((B, D), jnp.float32),
    )(selff, acc, msumg, W1, b12)
    return out


# speedup vs baseline: 1.7702x; 1.7702x over previous
import functools

import jax
import jax.numpy as jnp
from jax import lax
from jax.experimental import pallas as pl
from jax.experimental.pallas import tpu as pltpu
from jax.experimental.pallas import tpu_sc as plsc

B = 4096
V = 100000
D = 64
L = 50
R = 5

NEG = -1e30

NC = 2
NS = 16
NW = NC * NS
PB = B // NW

VB = 1000
MS = 16
CH = 4
LP = 56


def _prep_body(feat_ref, mask_ref, r_ref, wgv_ref, bgv_ref, rate_ref,
               fp_ref, rpp_ref, msum_ref, rext_ref):
    f = feat_ref[...]
    wt = wgv_ref[:D, :]
    wb = wgv_ref[D:, :]
    fp_ref[...] = jnp.dot(f, wt, preferred_element_type=jnp.float32)
    m = mask_ref[...]
    rpp_ref[...] = jnp.where(m > 0, r_ref[...], R)
    s = jnp.sum(m.astype(jnp.float32), axis=1, keepdims=True)
    msum_ref[...] = jnp.broadcast_to(s, (VB, MS))
    rex = jnp.dot(rate_ref[...], wb, preferred_element_type=jnp.float32) + bgv_ref[...]
    row = lax.broadcasted_iota(jnp.int32, (8, D), 0)
    rext_ref[...] = jnp.where(row < R, rex, NEG)


def _combine_body(self_ref, acc_ref, msum_ref, w1_ref, b1_ref, out_ref):
    denom = jnp.maximum(msum_ref[:, :1], 1.0)
    neigh = acc_ref[...] / denom
    x = (jnp.dot(self_ref[...], w1_ref[:D, :], preferred_element_type=jnp.float32)
         + jnp.dot(neigh, w1_ref[D:, :], preferred_element_type=jnp.float32)
         + b1_ref[...])
    out_ref[...] = jnp.maximum(x, 0.0)


def _sc_body(nodes_hbm, uv_hbm, rpp_hbm, fp_hbm, rext_hbm, msum_hbm, feat_hbm,
             acc_out, self_out, msum_out,
             nodes_v, uv_rows, rp_rows, msum_rows, self_rows, neigh_buf,
             e_buf, rr_buf,
             se0, se1, se2, se3, sr0, sr1, sr2, sr3, sem_a, sem_b):
    sem_e = [se0, se1, se2, se3]
    sem_r = [sr0, sr1, sr2, sr3]
    wid = lax.axis_index("s") * NC + lax.axis_index("c")
    base = wid * PB
    pltpu.sync_copy(nodes_hbm.at[pl.ds(base, PB)], nodes_v)
    cp_uv = pltpu.async_copy(uv_hbm.at[nodes_v], uv_rows, sem_e[0])
    cp_rp = pltpu.async_copy(rpp_hbm.at[nodes_v], rp_rows, sem_r[0])
    cp_ms = pltpu.async_copy(msum_hbm.at[nodes_v], msum_rows, sem_a)
    cp_sf = pltpu.async_copy(feat_hbm.at[nodes_v], self_rows, sem_b)
    cp_uv.wait()
    cp_rp.wait()

    def fire(g, buf):
        for j in range(CH):
            i = g * CH + j
            pltpu.async_copy(fp_hbm.at[uv_rows.at[i]], e_buf.at[buf, j], sem_e[j])
            pltpu.async_copy(rext_hbm.at[rp_rows.at[i]], rr_buf.at[buf, j], sem_r[j])

    def drain(g, buf):
        for j in range(CH):
            i = g * CH + j
            pltpu.make_async_copy(fp_hbm.at[uv_rows.at[i]], e_buf.at[buf, j], sem_e[j]).wait()
            pltpu.make_async_copy(rext_hbm.at[rp_rows.at[i]], rr_buf.at[buf, j], sem_r[j]).wait()

    def compute(g, buf):
        for j in range(CH):
            def lbody(l, accs):
                out = []
                for c in range(4):
                    e = e_buf[buf, j, l, pl.ds(c * 16, 16)]
                    rr = rr_buf[buf, j, l, pl.ds(c * 16, 16)]
                    out.append(accs[c] + jnp.maximum(e + rr, 0.0))
                return tuple(out)
            z = jnp.zeros((16,), jnp.float32)
            accs = lax.fori_loop(0, L, lbody, (z, z, z, z))
            for c in range(4):
                neigh_buf[g * CH + j, pl.ds(c * 16, 16)] = accs[c]

    NG = PB // CH
    fire(0, 0)

    def step(g, _):
        buf = lax.rem(g, 2)
        drain(g, buf)

        @pl.when(g < NG - 1)
        def _():
            fire(g + 1, 1 - buf)

        compute(g, buf)
        return 0

    lax.fori_loop(0, NG, step, 0)

    pltpu.sync_copy(neigh_buf, acc_out.at[pl.ds(base, PB)])
    cp_sf.wait()
    pltpu.sync_copy(self_rows, self_out.at[pl.ds(base, PB)])
    cp_ms.wait()
    pltpu.sync_copy(msum_rows, msum_out.at[pl.ds(base, PB)])


def kernel(nodes, hist_uv, hist_uv_mask, hist_r, hist_r_mask,
           features, rating_emb, W_gv, b_gv, W1, b1):
    nodes = nodes.astype(jnp.int32)
    pad = ((0, 0), (0, LP - L))
    uvp = jnp.pad(hist_uv.astype(jnp.int32), pad)
    hist_r = jnp.pad(hist_r.astype(jnp.int32), pad)
    mask = jnp.pad(hist_uv_mask.astype(jnp.int32), pad)
    rate_pad = jnp.zeros((8, D), jnp.float32).at[:R].set(rating_emb)
    bgv2 = b_gv.reshape(1, D)
    b12 = b1.reshape(1, D)

    grid1 = V // VB
    fp, rpp, msum, rext = pl.pallas_call(
        _prep_body,
        grid=(grid1,),
        in_specs=[
            pl.BlockSpec((VB, D), lambda i: (i, 0)),
            pl.BlockSpec((VB, LP), lambda i: (i, 0)),
            pl.BlockSpec((VB, LP), lambda i: (i, 0)),
            pl.BlockSpec((2 * D, D), lambda i: (0, 0)),
            pl.BlockSpec((1, D), lambda i: (0, 0)),
            pl.BlockSpec((8, D), lambda i: (0, 0)),
        ],
        out_specs=[
            pl.BlockSpec((VB, D), lambda i: (i, 0)),
            pl.BlockSpec((VB, LP), lambda i: (i, 0)),
            pl.BlockSpec((VB, MS), lambda i: (i, 0)),
            pl.BlockSpec((8, D), lambda i: (0, 0)),
        ],
        out_shape=[
            jax.ShapeDtypeStruct((V, D), jnp.float32),
            jax.ShapeDtypeStruct((V, LP), jnp.int32),
            jax.ShapeDtypeStruct((V, MS), jnp.float32),
            jax.ShapeDtypeStruct((8, D), jnp.float32),
        ],
    )(features, mask, hist_r, W_gv, bgv2, rate_pad)

    mesh = plsc.VectorSubcoreMesh(core_axis_name="c", subcore_axis_name="s")
    acc, selff, msumg = pl.kernel(
        _sc_body,
        out_type=[
            jax.ShapeDtypeStruct((B, D), jnp.float32),
            jax.ShapeDtypeStruct((B, D), jnp.float32),
            jax.ShapeDtypeStruct((B, MS), jnp.float32),
        ],
        mesh=mesh,
        compiler_params=pltpu.CompilerParams(use_tc_tiling_on_sc=False),
        scratch_types=[
            pltpu.VMEM((PB,), jnp.int32),
            pltpu.VMEM((PB, LP), jnp.int32),
            pltpu.VMEM((PB, LP), jnp.int32),
            pltpu.VMEM((PB, MS), jnp.float32),
            pltpu.VMEM((PB, D), jnp.float32),
            pltpu.VMEM((PB, D), jnp.float32),
            pltpu.VMEM((2, CH, LP, D), jnp.float32),
            pltpu.VMEM((2, CH, LP, D), jnp.float32),
        ] + [pltpu.SemaphoreType.DMA] * 10,
    )(nodes, uvp, rpp, fp, rext, msum, features)

    BBLK = 512
    out = pl.pallas_call(
        _combine_body,
        grid=(B // BBLK,),
        in_specs=[
            pl.BlockSpec((BBLK, D), lambda i: (i, 0)),
            pl.BlockSpec((BBLK, D), lambda i: (i, 0)),
            pl.BlockSpec((BBLK, MS), lambda i: (i, 0)),
            pl.BlockSpec((2 * D, D), lambda i: (0, 0)),
            pl.BlockSpec((1, D), lambda i: (0, 0)),
        ],
        out_specs=pl.BlockSpec((BBLK, D), lambda i: (i, 0)),
        out_shape=jax.ShapeDtypeStruct((B, D), jnp.float32),
    )(selff, acc, msumg, W1, b12)
    return out
